# baseline (speedup 1.0000x reference)
import functools

import jax
import jax.numpy as jnp
from jax import lax
from jax.experimental import pallas as pl
from jax.experimental.pallas import tpu as pltpu
from jax.experimental.pallas import tpu_sc as plsc

_VOCAB = 1000
_EMBED = 128
_CONTEXT = 4
_BATCH = 16384
_HIDDEN = 128

_NC, _NS = 2, 16
_NW = _NC * _NS

_CHUNKS = (4096, 12288)
_BB = 512


def _sc_gather(emb, idx):
    n = idx.shape[0]
    rows_per_w = n // _NW
    mesh = plsc.VectorSubcoreMesh(core_axis_name="c", subcore_axis_name="s")

    @functools.partial(
        pl.kernel,
        out_type=jax.ShapeDtypeStruct((n, _EMBED), jnp.float32),
        mesh=mesh,
        scratch_types=[
            pltpu.VMEM((rows_per_w,), jnp.int32),
            pltpu.VMEM((rows_per_w, _EMBED), jnp.float32),
            pltpu.SemaphoreType.DMA,
        ],
    )
    def body(emb_hbm, idx_hbm, out_hbm, idx_v, rows_v, sem):
        wid = lax.axis_index("s") * _NC + lax.axis_index("c")
        base = wid * rows_per_w
        pltpu.sync_copy(idx_hbm.at[pl.ds(base, rows_per_w)], idx_v)
        pltpu.async_copy(emb_hbm.at[idx_v], rows_v, sem).wait()
        pltpu.sync_copy(rows_v, out_hbm.at[pl.ds(base, rows_per_w)])

    return body(emb, idx)


def _mlp_body(e_ref, w1_ref, b1_ref, w2t_ref, b2_ref, out_ref):
    ht = lax.dot_general(w1_ref[...], e_ref[...], (((0,), (1,)), ((), ())),
                         preferred_element_type=jnp.float32)
    ht = jnp.maximum(ht + b1_ref[...], 0.0).astype(jnp.bfloat16)
    for c in range(_CONTEXT):
        out_ref[c] = (
            jnp.dot(w2t_ref[c], ht, preferred_element_type=jnp.float32)
            + b2_ref[c]
        )


def _mlp_body_alias(prev_ref, e_ref, w1_ref, b1_ref, w2t_ref, b2_ref, out_ref):
    del prev_ref
    _mlp_body(e_ref, w1_ref, b1_ref, w2t_ref, b2_ref, out_ref)


def _tc_mlp_chunk(e, w1, b1, w2t, b2, prev, base_block):
    nblocks = e.shape[0] // _BB
    in_specs = [
        pl.BlockSpec((_BB, _EMBED), lambda i: (i, 0)),
        pl.BlockSpec((_EMBED, _HIDDEN), lambda i: (0, 0)),
        pl.BlockSpec((_HIDDEN, 1), lambda i: (0, 0)),
        pl.BlockSpec((_CONTEXT, _VOCAB, _HIDDEN), lambda i: (0, 0, 0)),
        pl.BlockSpec((_CONTEXT, _VOCAB, 1), lambda i: (0, 0, 0)),
    ]
    args = [e, w1, b1, w2t, b2]
    io_alias = {}
    body = _mlp_body
    if prev is not None:
        in_specs = [pl.BlockSpec(memory_space=pl.ANY)] + in_specs
        args = [prev] + args
        io_alias = {0: 0}
        body = _mlp_body_alias
    return pl.pallas_call(
        body,
        grid=(nblocks,),
        in_specs=in_specs,
        out_specs=pl.BlockSpec((_CONTEXT, _VOCAB, _BB),
                               lambda i, b=base_block: (0, 0, b + i)),
        out_shape=jax.ShapeDtypeStruct((_CONTEXT, _VOCAB, _BATCH), jnp.float32),
        input_output_aliases=io_alias,
    )(*args)


def kernel(x, emb, W1, b1, W2, b2):
    xs = x.astype(jnp.int32)
    w2t = W2.T.reshape(_CONTEXT, _VOCAB, _HIDDEN).astype(jnp.bfloat16)
    b1c = b1.reshape(_HIDDEN, 1)
    b2c = b2.reshape(_CONTEXT, _VOCAB, 1)
    es, lo = [], 0
    for n in _CHUNKS:
        es.append(_sc_gather(emb, lax.slice(xs, (lo,), (lo + n,))))
        lo += n
    out, base = None, 0
    for e in es:
        out = _tc_mlp_chunk(e, W1, b1c, w2t, b2c, out, base)
        base += e.shape[0] // _BB
    return out.transpose(2, 0, 1)


# speedup vs baseline: 1.0477x; 1.0477x over previous
import functools

import jax
import jax.numpy as jnp
from jax import lax
from jax.experimental import pallas as pl
from jax.experimental.pallas import tpu as pltpu
from jax.experimental.pallas import tpu_sc as plsc

_VOCAB = 1000
_EMBED = 128
_CONTEXT = 4
_BATCH = 16384
_HIDDEN = 128

_NC, _NS = 2, 16
_NW = _NC * _NS
_ROWS_PER_W = _BATCH // _NW


def _sc_gather(emb, idx):
    mesh = plsc.VectorSubcoreMesh(core_axis_name="c", subcore_axis_name="s")

    @functools.partial(
        pl.kernel,
        out_type=jax.ShapeDtypeStruct((_BATCH, _EMBED), jnp.float32),
        mesh=mesh,
        scratch_types=[
            pltpu.VMEM((_ROWS_PER_W,), jnp.int32),
            pltpu.VMEM((_ROWS_PER_W, _EMBED), jnp.float32),
            pltpu.SemaphoreType.DMA,
        ],
    )
    def body(emb_hbm, idx_hbm, out_hbm, idx_v, rows_v, sem):
        wid = lax.axis_index("s") * _NC + lax.axis_index("c")
        base = wid * _ROWS_PER_W
        pltpu.sync_copy(idx_hbm.at[pl.ds(base, _ROWS_PER_W)], idx_v)
        pltpu.async_copy(emb_hbm.at[idx_v], rows_v, sem).wait()
        pltpu.sync_copy(rows_v, out_hbm.at[pl.ds(base, _ROWS_PER_W)])

    return body(emb, idx)


_BB = 1024


def _mlp_body(e_ref, w1_ref, b1_ref, w2t_ref, b2_ref, out_ref):
    ht = lax.dot_general(w1_ref[...], e_ref[...], (((0,), (1,)), ((), ())),
                         preferred_element_type=jnp.float32)
    ht = jnp.maximum(ht + b1_ref[...], 0.0).astype(jnp.bfloat16)
    for c in range(_CONTEXT):
        out_ref[c] = (
            jnp.dot(w2t_ref[c], ht, preferred_element_type=jnp.float32)
            + b2_ref[c]
        )


def _tc_mlp(e, w1, b1, w2t, b2):
    grid = (_BATCH // _BB,)
    return pl.pallas_call(
        _mlp_body,
        grid=grid,
        in_specs=[
            pl.BlockSpec((_BB, _EMBED), lambda i: (i, 0)),
            pl.BlockSpec((_EMBED, _HIDDEN), lambda i: (0, 0)),
            pl.BlockSpec((_HIDDEN, 1), lambda i: (0, 0)),
            pl.BlockSpec((_CONTEXT, _VOCAB, _HIDDEN), lambda i: (0, 0, 0)),
            pl.BlockSpec((_CONTEXT, _VOCAB, 1), lambda i: (0, 0, 0)),
        ],
        out_specs=pl.BlockSpec((_CONTEXT, _VOCAB, _BB), lambda i: (0, 0, i)),
        out_shape=jax.ShapeDtypeStruct((_CONTEXT, _VOCAB, _BATCH), jnp.float32),
    )(e, w1, b1, w2t, b2)


def kernel(x, emb, W1, b1, W2, b2):
    e = _sc_gather(emb, x.astype(jnp.int32))
    w2t = W2.T.reshape(_CONTEXT, _VOCAB, _HIDDEN).astype(jnp.bfloat16)
    out = _tc_mlp(e, W1, b1.reshape(_HIDDEN, 1), w2t,
                  b2.reshape(_CONTEXT, _VOCAB, 1))
    return out.transpose(2, 0, 1)
